# baseline (speedup 1.0000x reference)
import functools

import jax
import jax.numpy as jnp
from jax.experimental import pallas as pl
from jax.experimental.pallas import tpu as pltpu

_T = 5
_ALPHA = 0.5


def _body(sc_ref, like_ref, tri_ref, m2_ref, e_ref, s_ref, out_ref):
    like = like_ref[...]
    tri = tri_ref[...]
    cr = jnp.zeros_like(like)
    for t in range(1, _T):
        cr = jnp.where(tri == t, sc_ref[t], cr)
    w = like * cr

    wrep = jnp.dot(w.astype(jnp.bfloat16), e_ref[...],
                   preferred_element_type=jnp.float32)
    p = wrep * m2_ref[...]
    cu = jnp.dot(p.astype(jnp.bfloat16), s_ref[...],
                 preferred_element_type=jnp.float32)

    lane = jax.lax.broadcasted_iota(jnp.int32, cu.shape, 1)
    sysrow = jnp.zeros_like(cu)
    for t in range(1, _T):
        sysrow = jnp.where(lane == t, sc_ref[_T + t], sysrow)
    total = _ALPHA * sysrow + (1.0 - _ALPHA) * cu

    masked = jnp.where(lane < _T, total, -1e30)
    mx = jnp.max(masked, axis=1, keepdims=True)
    sh = masked - mx
    lse = jnp.log(jnp.sum(jnp.exp(sh), axis=1, keepdims=True))
    out_ref[...] = (sh - lse)[:, :8]


@functools.partial(jax.jit, static_argnums=())
def kernel(likelihoods, decision_mask, cruelty_parameters, system_parameters,
           disease_triages):
    B, D = likelihoods.shape
    T = decision_mask.shape[2]
    K = D * T
    bB = 512

    m2 = decision_mask.reshape(B, K)
    tri = disease_triages.astype(jnp.int32)
    cost = jnp.concatenate([jnp.zeros((1,), jnp.float32),
                            cruelty_parameters.astype(jnp.float32)])
    sysc = jnp.concatenate([jnp.zeros((1,), jnp.float32),
                            system_parameters.astype(jnp.float32)])
    scalars = jnp.concatenate([cost, sysc])

    e_mat = (jnp.arange(K)[None, :] // T == jnp.arange(D)[:, None]
             ).astype(jnp.bfloat16)
    s_mat = (jnp.arange(K)[:, None] % T == jnp.arange(128)[None, :]
             ).astype(jnp.bfloat16)

    out = pl.pallas_call(
        _body,
        grid=(B // bB,),
        in_specs=[
            pl.BlockSpec(memory_space=pltpu.SMEM),
            pl.BlockSpec((bB, D), lambda i: (i, 0)),
            pl.BlockSpec((bB, D), lambda i: (i, 0)),
            pl.BlockSpec((bB, K), lambda i: (i, 0)),
            pl.BlockSpec((D, K), lambda i: (0, 0)),
            pl.BlockSpec((K, 128), lambda i: (0, 0)),
        ],
        out_specs=pl.BlockSpec((bB, 8), lambda i: (i, 0)),
        out_shape=jax.ShapeDtypeStruct((B, 8), jnp.float32),
        compiler_params=pltpu.CompilerParams(
            dimension_semantics=("parallel",)),
    )(scalars, likelihoods, tri, m2, e_mat, s_mat)
    return out[:, :T]


# speedup vs baseline: 5.9835x; 5.9835x over previous
import functools

import jax
import jax.numpy as jnp
from jax.experimental import pallas as pl
from jax.experimental.pallas import tpu as pltpu

_T = 5
_ALPHA = 0.5


def _body(sc_ref, like_ref, tri_ref, m_ref, out_ref):
    like = like_ref[...]
    tri = tri_ref[...]
    cr = jnp.zeros_like(like)
    for t in range(1, _T):
        cr = jnp.where(tri == t, sc_ref[t], cr)
    w = like * cr

    totals = []
    for t in range(_T):
        cu = jnp.sum(w * m_ref[t], axis=1)
        totals.append(_ALPHA * sc_ref[_T + t] + (1.0 - _ALPHA) * cu)

    mx = totals[0]
    for t in range(1, _T):
        mx = jnp.maximum(mx, totals[t])
    exps = [jnp.exp(tt - mx) for tt in totals]
    s = exps[0]
    for t in range(1, _T):
        s = s + exps[t]
    lse = jnp.log(s)
    for t in range(_T):
        out_ref[t, :] = totals[t] - mx - lse


@functools.partial(jax.jit, static_argnums=())
def kernel(likelihoods, decision_mask, cruelty_parameters, system_parameters,
           disease_triages):
    B, D = likelihoods.shape
    T = decision_mask.shape[2]
    bB = 512

    mask_t = jnp.transpose(decision_mask, (2, 0, 1))
    tri = disease_triages.astype(jnp.int32)
    cost = jnp.concatenate([jnp.zeros((1,), jnp.float32),
                            cruelty_parameters.astype(jnp.float32)])
    sysc = jnp.concatenate([jnp.zeros((1,), jnp.float32),
                            system_parameters.astype(jnp.float32)])
    scalars = jnp.concatenate([cost, sysc])

    out = pl.pallas_call(
        _body,
        grid=(B // bB,),
        in_specs=[
            pl.BlockSpec(memory_space=pltpu.SMEM),
            pl.BlockSpec((bB, D), lambda i: (i, 0)),
            pl.BlockSpec((bB, D), lambda i: (i, 0)),
            pl.BlockSpec((T, bB, D), lambda i: (0, i, 0)),
        ],
        out_specs=pl.BlockSpec((T, bB), lambda i: (0, i)),
        out_shape=jax.ShapeDtypeStruct((T, B), jnp.float32),
        compiler_params=pltpu.CompilerParams(
            dimension_semantics=("parallel",)),
    )(scalars, likelihoods, tri, mask_t)
    return jnp.transpose(out, (1, 0))
